# baseline (speedup 1.0000x reference)
import numpy as np
import jax
import jax.numpy as jnp
from jax.experimental import pallas as pl
from jax.experimental.pallas import tpu as pltpu

_C_IN = 4
_H = 16
_W = 16
_OC = 8
_NCLS = 10
_BT = 512
_MB = _BT * _H
_LN = _OC * _W


def _shift_delta():
    d = np.zeros((3, _W, _W), np.float32)
    for dd in range(3):
        for w in range(_W):
            v = w - (dd - 1)
            if 0 <= v < _W:
                d[dd, w, v] = 1.0
    return d


_D_NP = _shift_delta()

_GM_NP = np.zeros((_H, _H * 16), np.float32)
for _h in range(_H):
    _GM_NP[_h, _h * 16:_h * 16 + _NCLS] = 1.0

_F_NP = np.tile(np.eye(16, dtype=np.float32)[:, :_NCLS], (_H, 1))


def _body(xr_ref, t1_ref, t23_ref, bl_ref, g2_ref, gm_ref, f_ref,
          hb_ref, o_ref):
    a = xr_ref[...]
    tw = [t1_ref[...], t23_ref[0], t23_ref[1]]
    for l in range(3):
        t = tw[l]
        p = jnp.dot(a, t, preferred_element_type=jnp.float32)
        y = jnp.concatenate([
            p[:_BT, _LN:2 * _LN] + p[_BT:2 * _BT, 2 * _LN:],
            (p[_BT:_MB - _BT, _LN:2 * _LN] + p[:_MB - 2 * _BT, :_LN]
             + p[2 * _BT:, 2 * _LN:]),
            p[_MB - _BT:, _LN:2 * _LN] + p[_MB - 2 * _BT:_MB - _BT, :_LN],
        ], axis=0) + bl_ref[l:l + 1, :]
        a = jnp.maximum(y, jnp.exp(jnp.minimum(y, 0.0)) - 1.0
                        ).astype(jnp.bfloat16)
    q = jnp.dot(a, g2_ref[...], preferred_element_type=jnp.float32)
    zt = [q[h * _BT:(h + 1) * _BT, :] * gm_ref[h:h + 1, :]
          for h in range(_H)]
    while len(zt) > 1:
        zt = [zt[i] + zt[i + 1] for i in range(0, len(zt), 2)]
    z = zt[0]
    logits = jnp.dot(z, f_ref[...], preferred_element_type=jnp.float32)
    logits = logits + hb_ref[...]
    m = jnp.max(logits, axis=1, keepdims=True)
    zc = logits - m
    lse = jnp.log(jnp.sum(jnp.exp(zc), axis=1, keepdims=True))
    o_ref[...] = (zc - lse).astype(o_ref.dtype)


def _run_chunk(x, params):
    t1, t23, bl, g2, gm, f, hb = params
    g = x.shape[0] // _BT
    xr = x.astype(jnp.bfloat16).reshape(g, _BT, _C_IN, _H, _W)
    xr = jnp.transpose(xr, (0, 3, 1, 2, 4)).reshape(g * _MB, _C_IN * _W)
    gh = g
    return pl.pallas_call(
        _body,
        out_shape=jax.ShapeDtypeStruct((g * _BT, _NCLS), jnp.float32),
        grid=(1, gh),
        in_specs=[
            pl.BlockSpec((_MB, _C_IN * _W), lambda i, j: (i * gh + j, 0)),
            pl.BlockSpec((_C_IN * _W, 3 * _LN), lambda i, j: (0, 0)),
            pl.BlockSpec((2, _LN, 3 * _LN), lambda i, j: (0, 0, 0)),
            pl.BlockSpec((3, _LN), lambda i, j: (0, 0)),
            pl.BlockSpec((_LN, _H * 16), lambda i, j: (0, 0)),
            pl.BlockSpec((_H, _H * 16), lambda i, j: (0, 0)),
            pl.BlockSpec((_H * 16, _NCLS), lambda i, j: (0, 0)),
            pl.BlockSpec((1, _NCLS), lambda i, j: (0, 0)),
        ],
        out_specs=pl.BlockSpec((_BT, _NCLS), lambda i, j: (i * gh + j, 0)),
        compiler_params=pltpu.CompilerParams(
            dimension_semantics=("arbitrary", "arbitrary"),
            vmem_limit_bytes=64 * 1024 * 1024),
    )(xr, t1, t23, bl, g2, gm, f, hb)


@jax.jit
def _forward(x, conv_w, conv_b, head_m, head_b):
    n = x.shape[0]
    g = 2 * pl.cdiv(n, 2 * _BT)
    n_pad = g * _BT

    x = x.astype(jnp.float32)
    if n_pad != n:
        x = jnp.pad(x, ((0, n_pad - n), (0, 0), (0, 0), (0, 0)))

    d = jnp.asarray(_D_NP)
    ts = []
    for l in range(3):
        wl4 = conv_w[l].reshape(_OC, 3, 3, _OC)
        t_l = jnp.einsum('cjdi,dwv->jiwcv', wl4, d)
        t_l = t_l.reshape(3, _OC * _W, _OC * _W)
        if l == 0:
            t_l = t_l[:, :_C_IN * _W, :]
        ts.append(jnp.concatenate([t_l[0], t_l[1], t_l[2]], axis=1))
    t1 = ts[0].astype(jnp.bfloat16)
    t23 = jnp.stack(ts[1:], axis=0).astype(jnp.bfloat16)
    bl = jnp.repeat(conv_b[:, :, 0], _W, axis=1)

    hm = head_m[:, :, :_H * _W].reshape(_OC, _NCLS, _H, _W)
    g2 = jnp.transpose(hm, (0, 3, 2, 1))
    g2 = jnp.pad(g2, ((0, 0), (0, 0), (0, 0), (0, 16 - _NCLS)))
    g2 = g2.reshape(_OC * _W, _H * 16).astype(jnp.bfloat16)

    gm = jnp.asarray(_GM_NP)
    f = jnp.asarray(_F_NP)
    hb = head_b[:, 0].reshape(1, _NCLS)

    params = (t1, t23, bl, g2, gm, f, hb)
    half = n_pad // 2
    out = jnp.concatenate([_run_chunk(x[:half], params),
                           _run_chunk(x[half:], params)], axis=0)
    return out[:n]


def kernel(x, conv_w, conv_b, head_m, head_b, masks, sel):
    del masks, sel
    return _forward(x, conv_w, conv_b, head_m, head_b)


# speedup vs baseline: 1.0673x; 1.0673x over previous
import numpy as np
import jax
import jax.numpy as jnp
from jax.experimental import pallas as pl
from jax.experimental.pallas import tpu as pltpu

_C_IN = 4
_H = 16
_W = 16
_OC = 8
_NCLS = 10
_BT = 1024
_MB = _BT * _H
_LN = _OC * _W


def _shift_delta():
    d = np.zeros((3, _W, _W), np.float32)
    for dd in range(3):
        for w in range(_W):
            v = w - (dd - 1)
            if 0 <= v < _W:
                d[dd, w, v] = 1.0
    return d


_D_NP = _shift_delta()

_GM_NP = np.zeros((_H, _H * 16), np.float32)
for _h in range(_H):
    _GM_NP[_h, _h * 16:_h * 16 + _NCLS] = 1.0

_F_NP = np.tile(np.eye(16, dtype=np.float32)[:, :_NCLS], (_H, 1))


def _body(xr_ref, t1_ref, t23_ref, bl_ref, g2_ref, gm_ref, f_ref,
          hb_ref, o_ref):
    a = xr_ref[...]
    tw = [t1_ref[...], t23_ref[0], t23_ref[1]]
    for l in range(3):
        t = tw[l]
        p = jnp.dot(a, t, preferred_element_type=jnp.float32)
        y = jnp.concatenate([
            p[:_BT, _LN:2 * _LN] + p[_BT:2 * _BT, 2 * _LN:],
            (p[_BT:_MB - _BT, _LN:2 * _LN] + p[:_MB - 2 * _BT, :_LN]
             + p[2 * _BT:, 2 * _LN:]),
            p[_MB - _BT:, _LN:2 * _LN] + p[_MB - 2 * _BT:_MB - _BT, :_LN],
        ], axis=0) + bl_ref[l:l + 1, :]
        a = jnp.maximum(y, jnp.exp(jnp.minimum(y, 0.0)) - 1.0
                        ).astype(jnp.bfloat16)
    q = jnp.dot(a, g2_ref[...], preferred_element_type=jnp.float32)
    zt = [q[h * _BT:(h + 1) * _BT, :] * gm_ref[h:h + 1, :]
          for h in range(_H)]
    while len(zt) > 1:
        zt = [zt[i] + zt[i + 1] for i in range(0, len(zt), 2)]
    z = zt[0]
    logits = jnp.dot(z, f_ref[...], preferred_element_type=jnp.float32)
    logits = logits + hb_ref[...]
    m = jnp.max(logits, axis=1, keepdims=True)
    zc = logits - m
    lse = jnp.log(jnp.sum(jnp.exp(zc), axis=1, keepdims=True))
    o_ref[...] = (zc - lse).astype(o_ref.dtype)


def _run_chunk(x, params):
    t1, t23, bl, g2, gm, f, hb = params
    g = x.shape[0] // _BT
    xr = x.astype(jnp.bfloat16).reshape(g, _BT, _C_IN, _H, _W)
    xr = jnp.transpose(xr, (0, 3, 1, 2, 4)).reshape(g * _MB, _C_IN * _W)
    gh = g
    return pl.pallas_call(
        _body,
        out_shape=jax.ShapeDtypeStruct((g * _BT, _NCLS), jnp.float32),
        grid=(1, gh),
        in_specs=[
            pl.BlockSpec((_MB, _C_IN * _W), lambda i, j: (i * gh + j, 0)),
            pl.BlockSpec((_C_IN * _W, 3 * _LN), lambda i, j: (0, 0)),
            pl.BlockSpec((2, _LN, 3 * _LN), lambda i, j: (0, 0, 0)),
            pl.BlockSpec((3, _LN), lambda i, j: (0, 0)),
            pl.BlockSpec((_LN, _H * 16), lambda i, j: (0, 0)),
            pl.BlockSpec((_H, _H * 16), lambda i, j: (0, 0)),
            pl.BlockSpec((_H * 16, _NCLS), lambda i, j: (0, 0)),
            pl.BlockSpec((1, _NCLS), lambda i, j: (0, 0)),
        ],
        out_specs=pl.BlockSpec((_BT, _NCLS), lambda i, j: (i * gh + j, 0)),
        compiler_params=pltpu.CompilerParams(
            dimension_semantics=("arbitrary", "arbitrary"),
            vmem_limit_bytes=64 * 1024 * 1024),
    )(xr, t1, t23, bl, g2, gm, f, hb)


@jax.jit
def _forward(x, conv_w, conv_b, head_m, head_b):
    n = x.shape[0]
    g = 2 * pl.cdiv(n, 2 * _BT)
    n_pad = g * _BT

    x = x.astype(jnp.float32)
    if n_pad != n:
        x = jnp.pad(x, ((0, n_pad - n), (0, 0), (0, 0), (0, 0)))

    d = jnp.asarray(_D_NP)
    ts = []
    for l in range(3):
        wl4 = conv_w[l].reshape(_OC, 3, 3, _OC)
        t_l = jnp.einsum('cjdi,dwv->jiwcv', wl4, d)
        t_l = t_l.reshape(3, _OC * _W, _OC * _W)
        if l == 0:
            t_l = t_l[:, :_C_IN * _W, :]
        ts.append(jnp.concatenate([t_l[0], t_l[1], t_l[2]], axis=1))
    t1 = ts[0].astype(jnp.bfloat16)
    t23 = jnp.stack(ts[1:], axis=0).astype(jnp.bfloat16)
    bl = jnp.repeat(conv_b[:, :, 0], _W, axis=1)

    hm = head_m[:, :, :_H * _W].reshape(_OC, _NCLS, _H, _W)
    g2 = jnp.transpose(hm, (0, 3, 2, 1))
    g2 = jnp.pad(g2, ((0, 0), (0, 0), (0, 0), (0, 16 - _NCLS)))
    g2 = g2.reshape(_OC * _W, _H * 16).astype(jnp.bfloat16)

    gm = jnp.asarray(_GM_NP)
    f = jnp.asarray(_F_NP)
    hb = head_b[:, 0].reshape(1, _NCLS)

    out = _run_chunk(x, (t1, t23, bl, g2, gm, f, hb))
    return out[:n]


def kernel(x, conv_w, conv_b, head_m, head_b, masks, sel):
    del masks, sel
    return _forward(x, conv_w, conv_b, head_m, head_b)
